# baseline (speedup 1.0000x reference)
import jax
import jax.numpy as jnp
from jax.experimental import pallas as pl
from jax.experimental.pallas import tpu as pltpu

_BITS = 8
_LEVELS = 2 ** _BITS - 1
_HALF = 2 ** (_BITS - 1)


def _minmax_body(x_ref, mn_ref, mx_ref):
    i = pl.program_id(0)
    bmn = jnp.min(x_ref[...])
    bmx = jnp.max(x_ref[...])

    @pl.when(i == 0)
    def _init():
        mn_ref[0] = bmn
        mx_ref[0] = bmx

    @pl.when(i != 0)
    def _acc():
        mn_ref[0] = jnp.minimum(mn_ref[0], bmn)
        mx_ref[0] = jnp.maximum(mx_ref[0], bmx)


def _quant_body(ms_ref, x_ref, q_ref):
    mn = ms_ref[0]
    inv_step = ms_ref[1]
    q_ref[...] = jnp.round(
        (x_ref[...] - mn) * inv_step - float(_HALF)
    ).astype(jnp.int8)


def kernel(input):
    shape = input.shape
    C = shape[-1]
    R = 1
    for s in shape[:-1]:
        R *= s
    x = input.reshape(R, C)

    nb1 = 32
    bs1 = R // nb1
    mn, mx = pl.pallas_call(
        _minmax_body,
        grid=(nb1,),
        in_specs=[pl.BlockSpec((bs1, C), lambda i: (i, 0))],
        out_specs=[
            pl.BlockSpec(memory_space=pltpu.SMEM),
            pl.BlockSpec(memory_space=pltpu.SMEM),
        ],
        out_shape=[
            jax.ShapeDtypeStruct((1,), jnp.float32),
            jax.ShapeDtypeStruct((1,), jnp.float32),
        ],
        compiler_params=pltpu.CompilerParams(
            dimension_semantics=("arbitrary",),
        ),
    )(x)

    mn_s = mn[0]
    step = (mx[0] - mn_s) / _LEVELS
    ms = jnp.stack([mn_s, 1.0 / step])

    nb2 = 32
    bs2 = R // nb2
    q = pl.pallas_call(
        _quant_body,
        grid=(nb2,),
        in_specs=[
            pl.BlockSpec(memory_space=pltpu.SMEM),
            pl.BlockSpec((bs2, C), lambda i: (i, 0)),
        ],
        out_specs=pl.BlockSpec((bs2, C), lambda i: (i, 0)),
        out_shape=jax.ShapeDtypeStruct((R, C), jnp.int8),
        compiler_params=pltpu.CompilerParams(
            dimension_semantics=("arbitrary",),
        ),
    )(ms, x)

    min_step = jnp.stack([mn_s, step])
    return (input, q.reshape(shape), min_step)


# speedup vs baseline: 1.3026x; 1.3026x over previous
import jax
import jax.numpy as jnp
from jax.experimental import pallas as pl
from jax.experimental.pallas import tpu as pltpu

_BITS = 8
_LEVELS = float(2 ** _BITS - 1)
_HALF = float(2 ** (_BITS - 1))


def _body(x_ref, q_ref, xc_ref, ms_ref, inv_ref):
    p = pl.program_id(0)
    j = pl.program_id(1)

    @pl.when(p == 0)
    def _phase_minmax():
        bmn = jnp.min(x_ref[...])
        bmx = jnp.max(x_ref[...])

        @pl.when(j == 0)
        def _init():
            ms_ref[0] = bmn
            ms_ref[1] = bmx

        @pl.when(j != 0)
        def _acc():
            ms_ref[0] = jnp.minimum(ms_ref[0], bmn)
            ms_ref[1] = jnp.maximum(ms_ref[1], bmx)

    @pl.when(p == 1)
    def _phase_quant():
        @pl.when(j == 0)
        def _finalize():
            step = (ms_ref[1] - ms_ref[0]) / _LEVELS
            ms_ref[1] = step
            inv_ref[0] = 1.0 / step

        x = x_ref[...]
        q_ref[...] = jnp.round(
            (x - ms_ref[0]) * inv_ref[0] - _HALF
        ).astype(jnp.int8)
        xc_ref[...] = x


def kernel(input):
    shape = input.shape
    C = shape[-1]
    R = 1
    for s in shape[:-1]:
        R *= s
    x = input.reshape(R, C)

    nb = 32
    bs = R // nb

    q, xc, ms = pl.pallas_call(
        _body,
        grid=(2, nb),
        in_specs=[pl.BlockSpec((bs, C), lambda p, j: (j, 0))],
        out_specs=[
            pl.BlockSpec((bs, C), lambda p, j: (jnp.where(p == 0, 0, j), 0)),
            pl.BlockSpec((bs, C), lambda p, j: (jnp.where(p == 0, 0, j), 0)),
            pl.BlockSpec(memory_space=pltpu.SMEM),
        ],
        out_shape=[
            jax.ShapeDtypeStruct((R, C), jnp.int8),
            jax.ShapeDtypeStruct((R, C), jnp.float32),
            jax.ShapeDtypeStruct((2,), jnp.float32),
        ],
        scratch_shapes=[pltpu.SMEM((1,), jnp.float32)],
        compiler_params=pltpu.CompilerParams(
            dimension_semantics=("arbitrary", "arbitrary"),
        ),
    )(x)

    return (xc.reshape(shape), q.reshape(shape), ms)


# speedup vs baseline: 1.4410x; 1.1063x over previous
import jax
import jax.numpy as jnp
from jax.experimental import pallas as pl
from jax.experimental.pallas import tpu as pltpu

_BITS = 8
_LEVELS = float(2 ** _BITS - 1)
_HALF = float(2 ** (_BITS - 1))


def _body(x_ref, q_ref, xc_ref, ms_ref, inv_ref):
    p = pl.program_id(0)
    j = pl.program_id(1)

    @pl.when(p == 0)
    def _phase_minmax():
        bmn = jnp.min(x_ref[...])
        bmx = jnp.max(x_ref[...])

        @pl.when(j == 0)
        def _init():
            ms_ref[0] = bmn
            ms_ref[1] = bmx

        @pl.when(j != 0)
        def _acc():
            ms_ref[0] = jnp.minimum(ms_ref[0], bmn)
            ms_ref[1] = jnp.maximum(ms_ref[1], bmx)

    @pl.when(p == 1)
    def _phase_quant():
        @pl.when(j == 0)
        def _finalize():
            step = (ms_ref[1] - ms_ref[0]) / _LEVELS
            ms_ref[1] = step
            inv_ref[0] = 1.0 / step

        x = x_ref[...]
        q_ref[...] = jnp.round(
            (x - ms_ref[0]) * inv_ref[0] - _HALF
        ).astype(jnp.int8)
        xc_ref[...] = x


def kernel(input):
    shape = input.shape
    C = shape[-1]
    R = 1
    for s in shape[:-1]:
        R *= s
    x = input.reshape(R, C)

    nb = 16
    bs = R // nb

    q, xc, ms = pl.pallas_call(
        _body,
        grid=(2, nb),
        in_specs=[pl.BlockSpec((bs, C), lambda p, j: (j, 0))],
        out_specs=[
            pl.BlockSpec((bs, C), lambda p, j: (jnp.where(p == 0, 0, j), 0)),
            pl.BlockSpec((bs, C), lambda p, j: (jnp.where(p == 0, 0, j), 0)),
            pl.BlockSpec(memory_space=pltpu.SMEM),
        ],
        out_shape=[
            jax.ShapeDtypeStruct((R, C), jnp.int8),
            jax.ShapeDtypeStruct((R, C), jnp.float32),
            jax.ShapeDtypeStruct((2,), jnp.float32),
        ],
        scratch_shapes=[pltpu.SMEM((1,), jnp.float32)],
        compiler_params=pltpu.CompilerParams(
            dimension_semantics=("arbitrary", "arbitrary"),
        ),
    )(x)

    return (xc.reshape(shape), q.reshape(shape), ms)
